# Initial kernel scaffold; baseline (speedup 1.0000x reference)
import jax
import jax.numpy as jnp
from jax.experimental import pallas as pl


def kernel(input, codes, codebooks, scales, bias):
    raise NotImplementedError("write your pallas kernel here")


# speedup vs baseline: 23.5297x; 23.5297x over previous
import functools

import jax
import jax.numpy as jnp
from jax import lax
from jax.experimental import pallas as pl
from jax.experimental.pallas import tpu as pltpu
from jax.experimental.pallas import tpu_sc as plsc

IN_F = 4096
OUT_F = 4096
GS = 8
GROUPS = IN_F // GS
NCB = 2
CB_SIZE = 2 ** 16
IDX_PER_ROW = GROUPS * NCB
NCHUNK = IDX_PER_ROW // 128
NC, NS = 2, 16
NW = NC * NS
ROWS_PER_W = OUT_F // NW


def _sc_gather_body(table_hbm, idx_hbm, wx_hbm, idx_v, rows_v, sem):
    wid = lax.axis_index("s") * NC + lax.axis_index("c")

    def row_body(i, carry):
        o = wid * ROWS_PER_W + i
        pltpu.sync_copy(idx_hbm.at[o], idx_v)
        copies = [
            pltpu.async_copy(
                table_hbm.at[idx_v.at[k]],
                rows_v.at[pl.ds(k * 128, 128)],
                sem,
            )
            for k in range(NCHUNK)
        ]
        for cp in copies:
            cp.wait()
        pltpu.sync_copy(rows_v, wx_hbm.at[o])
        return carry

    lax.fori_loop(0, ROWS_PER_W, row_body, 0)


@jax.jit
def _sc_gather(table, idx):
    mesh = plsc.VectorSubcoreMesh(core_axis_name="c", subcore_axis_name="s")
    f = functools.partial(
        pl.kernel,
        out_type=jax.ShapeDtypeStruct((OUT_F, IDX_PER_ROW, GS), jnp.float32),
        mesh=mesh,
        scratch_types=[
            pltpu.VMEM((NCHUNK, 128), jnp.int32),
            pltpu.VMEM((IDX_PER_ROW, GS), jnp.float32),
            pltpu.SemaphoreType.DMA,
        ],
        compiler_params=pltpu.CompilerParams(use_tc_tiling_on_sc=False),
    )(_sc_gather_body)
    return f(table, idx)


def _mm_body(x_ref, w_ref, s_ref, b_ref, o_ref):
    acc = lax.dot_general(
        x_ref[...],
        w_ref[...],
        (((1,), (1,)), ((), ())),
        preferred_element_type=jnp.float32,
    )
    o_ref[...] = acc * s_ref[...] + b_ref[...]


@jax.jit
def _tc_matmul(x2, wcat, scales_row, bias_row):
    bn = 256
    grid = OUT_F // bn
    return pl.pallas_call(
        _mm_body,
        grid=(grid,),
        in_specs=[
            pl.BlockSpec((32, 2 * IN_F), lambda j: (0, 0)),
            pl.BlockSpec((bn, 2 * IN_F), lambda j: (j, 0)),
            pl.BlockSpec((1, bn), lambda j: (0, j)),
            pl.BlockSpec((1, bn), lambda j: (0, j)),
        ],
        out_specs=pl.BlockSpec((32, bn), lambda j: (0, j)),
        out_shape=jax.ShapeDtypeStruct((32, OUT_F), jnp.float32),
    )(x2, wcat, scales_row, bias_row)


def kernel(input, codes, codebooks, scales, bias):
    table = codebooks.reshape(NCB * CB_SIZE, GS)
    idx = jnp.concatenate(
        [codes[:, :, 0], codes[:, :, 1] + CB_SIZE], axis=1
    ).reshape(OUT_F, NCHUNK, 128)
    wcat = _sc_gather(table, idx).reshape(OUT_F, NCB * IN_F)
    x2 = jnp.concatenate([input, input], axis=1)
    return _tc_matmul(
        x2, wcat, scales.reshape(1, OUT_F), bias.reshape(1, OUT_F)
    )
